# baseline (speedup 1.0000x reference)
import jax
import jax.numpy as jnp
from jax.experimental import pallas as pl
from jax.experimental.pallas import tpu as pltpu

_B, _IN, _H = 128, 512, 1024
_G = 2
_CH = 256

_NT = (((1,), (1,)), ((), ()))
_OUTER = (((0,), (0,)), ((), ()))


def _main_body(plastic_ref, items_ref, reward_ref, W1_ref, b1_ref, W2_ref,
               b2_ref, Wr_ref, br_ref, alpha_ref, W3_ref, b3_ref, Wn_ref,
               bn_ref, Wc_ref, bc_ref, Wv_ref, bv_ref,
               choice_ref, nm_ref, value_ref, newp_ref, hid_ref):
    G, H, CH = _G, _H, _CH
    bf16 = jnp.bfloat16
    emb = jnp.tanh(
        jax.lax.dot_general(items_ref[...].reshape(G, -1).astype(bf16), W1_ref[...], _NT,
                            preferred_element_type=jnp.float32) + b1_ref[...])
    embb = emb.astype(bf16)
    cur = jax.lax.dot_general(embb, W2_ref[...], _NT,
                              preferred_element_type=jnp.float32) + b2_ref[...]
    rew = reward_ref[...].reshape(G, 1) * Wr_ref[...] + br_ref[...]
    pres = []
    for g in range(G):
        parts = []
        for c in range(0, H, CH):
            M = alpha_ref[c:c + CH, :] * plastic_ref[g, c:c + CH, :].astype(bf16)
            parts.append(jax.lax.dot_general(embb[g:g + 1], M, _NT,
                                             preferred_element_type=jnp.float32))
        pres.append(jnp.concatenate(parts, axis=1))
    pre = jnp.concatenate(pres, axis=0) + cur
    hidden = jnp.tanh(pre)
    hid_ref[...] = hidden.reshape(G, 1, H)
    t = (hidden + rew).astype(bf16)
    h3 = jnp.tanh(
        jax.lax.dot_general(t, W3_ref[...], _NT,
                            preferred_element_type=jnp.float32) + b3_ref[...])
    nmo = jnp.tanh(
        jax.lax.dot_general(h3.astype(bf16), Wn_ref[...], _NT,
                            preferred_element_type=jnp.float32) + bn_ref[...])
    nm = nmo[:, 0:1] - nmo[:, 1:2]
    nm_ref[...] = nm.reshape(G, 1, 1)
    choice_ref[...] = jax.nn.sigmoid(
        jnp.sum(hidden * Wc_ref[...], axis=1, keepdims=True)
        + bc_ref[...]).reshape(G, 1, 1)
    value_ref[...] = (jnp.sum(hidden * Wv_ref[...], axis=1, keepdims=True)
                      + bv_ref[...]).reshape(G, 1, 1)
    nm10 = nm * 10.0
    for g in range(G):
        for c in range(0, H, CH):
            outer = jax.lax.dot_general(pre[g:g + 1, c:c + CH], emb[g:g + 1],
                                        _OUTER, preferred_element_type=jnp.float32)
            newp_ref[g, c:c + CH, :] = jnp.clip(
                plastic_ref[g, c:c + CH, :] + nm10[g:g + 1] * jnp.tanh(outer),
                -50.0, 50.0)


def kernel(items, plastic_weights, reward, W1, b1, W2, b2, W3, b3,
           Wc, bc, Wr, br, Wn, bn, alpha, Wv, bv):
    f32 = jnp.float32
    bf16 = jnp.bfloat16
    B, IN, H, G = _B, _IN, _H, _G

    row3 = pl.BlockSpec((G, 1, 1), lambda b: (b, 0, 0))
    full = lambda shape: pl.BlockSpec(shape, lambda b: tuple(0 for _ in shape))

    choice3, nm3, value3, newp, hid3 = pl.pallas_call(
        _main_body,
        grid=(B // G,),
        in_specs=[
            pl.BlockSpec((G, H, H), lambda b: (b, 0, 0)),
            pl.BlockSpec((G, 1, IN), lambda b: (b, 0, 0)),
            row3,
            full((H, IN)),
            full((1, H)),
            full((H, H)),
            full((1, H)),
            full((1, H)),
            full((1, H)),
            full((H, H)),
            full((H, H)),
            full((1, H)),
            full((2, H)),
            full((1, 2)),
            full((1, H)),
            full((1, 1)),
            full((1, H)),
            full((1, 1)),
        ],
        out_specs=[
            pl.BlockSpec((G, 1, 1), lambda b: (b, 0, 0)),
            pl.BlockSpec((G, 1, 1), lambda b: (b, 0, 0)),
            pl.BlockSpec((G, 1, 1), lambda b: (b, 0, 0)),
            pl.BlockSpec((G, H, H), lambda b: (b, 0, 0)),
            pl.BlockSpec((G, 1, H), lambda b: (b, 0, 0)),
        ],
        out_shape=[
            jax.ShapeDtypeStruct((B, 1, 1), f32),
            jax.ShapeDtypeStruct((B, 1, 1), f32),
            jax.ShapeDtypeStruct((B, 1, 1), f32),
            jax.ShapeDtypeStruct((B, H, H), f32),
            jax.ShapeDtypeStruct((B, 1, H), f32),
        ],
        compiler_params=pltpu.CompilerParams(
            dimension_semantics=("parallel",),
            vmem_limit_bytes=100 * 1024 * 1024),
    )(plastic_weights, items.reshape(B, 1, IN), reward.reshape(B, 1, 1),
      W1.astype(bf16), b1.reshape(1, H), W2.astype(bf16), b2.reshape(1, H),
      Wr.reshape(1, H), br.reshape(1, H), alpha.astype(bf16),
      W3.astype(bf16), b3.reshape(1, H), Wn.astype(bf16), bn.reshape(1, 2),
      Wc, bc.reshape(1, 1), Wv, bv.reshape(1, 1))

    return (choice3.reshape(B, 1), nm3, value3.reshape(B, 1),
            newp, hid3.reshape(B, H))


# speedup vs baseline: 1.0304x; 1.0304x over previous
import jax
import jax.numpy as jnp
from jax.experimental import pallas as pl
from jax.experimental.pallas import tpu as pltpu

_B, _IN, _H = 128, 512, 1024
_G = 2

_NT = (((1,), (1,)), ((), ()))
_OUTER = (((0,), (0,)), ((), ()))


def _prologue_body(items_ref, reward_ref, W1_ref, b1_ref, W2_ref, b2_ref,
                   Wr_ref, br_ref, emb_ref, cur_ref, rew_ref):
    emb = jnp.tanh(
        jax.lax.dot_general(items_ref[...], W1_ref[...], _NT,
                            preferred_element_type=jnp.float32) + b1_ref[...])
    emb_ref[...] = emb
    cur_ref[...] = jax.lax.dot_general(
        emb, W2_ref[...], _NT, preferred_element_type=jnp.float32) + b2_ref[...]
    rew_ref[...] = reward_ref[...] * Wr_ref[...] + br_ref[...]


def _main_body(plastic_ref, emb_ref, cur_ref, rew_ref, alpha_ref, W3_ref,
               b3_ref, Wn_ref, bn_ref, Wc_ref, bc_ref, Wv_ref, bv_ref,
               choice_ref, nm_ref, value_ref, newp_ref, hid_ref):
    G, H = _G, _H
    alpha = alpha_ref[...]
    emb = emb_ref[...].reshape(G, H)
    pres = []
    for g in range(G):
        M = alpha * plastic_ref[g]
        contrib = jax.lax.dot_general(emb[g:g + 1], M, _NT,
                                      preferred_element_type=jnp.float32)
        pres.append(contrib)
    pre = jnp.concatenate(pres, axis=0) + cur_ref[...].reshape(G, H)
    hidden = jnp.tanh(pre)
    hid_ref[...] = hidden.reshape(G, 1, H)
    t = hidden + rew_ref[...].reshape(G, H)
    h3 = jnp.tanh(
        jax.lax.dot_general(t, W3_ref[...], _NT,
                            preferred_element_type=jnp.float32) + b3_ref[...])
    nmo = jnp.tanh(
        jax.lax.dot_general(h3, Wn_ref[...], _NT,
                            preferred_element_type=jnp.float32) + bn_ref[...])
    nm = nmo[:, 0:1] - nmo[:, 1:2]
    nm_ref[...] = nm.reshape(G, 1, 1)
    choice_ref[...] = jax.nn.sigmoid(
        jnp.sum(hidden * Wc_ref[...], axis=1, keepdims=True)
        + bc_ref[...]).reshape(G, 1, 1)
    value_ref[...] = (jnp.sum(hidden * Wv_ref[...], axis=1, keepdims=True)
                      + bv_ref[...]).reshape(G, 1, 1)
    nm10 = nm * 10.0
    for g in range(G):
        outer = jax.lax.dot_general(pre[g:g + 1], emb[g:g + 1], _OUTER,
                                    preferred_element_type=jnp.float32)
        newp_ref[g] = jnp.clip(
            plastic_ref[g] + nm10[g:g + 1] * jnp.tanh(outer), -50.0, 50.0)


def kernel(items, plastic_weights, reward, W1, b1, W2, b2, W3, b3,
           Wc, bc, Wr, br, Wn, bn, alpha, Wv, bv):
    f32 = jnp.float32
    B, IN, H, G = _B, _IN, _H, _G
    half = B // 2

    emb, cur, rew = pl.pallas_call(
        _prologue_body,
        grid=(2,),
        in_specs=[
            pl.BlockSpec((half, IN), lambda i: (i, 0)),
            pl.BlockSpec((half, 1), lambda i: (i, 0)),
            pl.BlockSpec((H, IN), lambda i: (0, 0)),
            pl.BlockSpec((1, H), lambda i: (0, 0)),
            pl.BlockSpec((H, H), lambda i: (0, 0)),
            pl.BlockSpec((1, H), lambda i: (0, 0)),
            pl.BlockSpec((1, H), lambda i: (0, 0)),
            pl.BlockSpec((1, H), lambda i: (0, 0)),
        ],
        out_specs=[
            pl.BlockSpec((half, H), lambda i: (i, 0)),
            pl.BlockSpec((half, H), lambda i: (i, 0)),
            pl.BlockSpec((half, H), lambda i: (i, 0)),
        ],
        out_shape=[jax.ShapeDtypeStruct((B, H), f32)] * 3,
        compiler_params=pltpu.CompilerParams(
            dimension_semantics=("parallel",)),
    )(items, reward, W1, b1.reshape(1, H), W2, b2.reshape(1, H),
      Wr.reshape(1, H), br.reshape(1, H))

    emb3 = emb.reshape(B, 1, H)
    cur3 = cur.reshape(B, 1, H)
    rew3 = rew.reshape(B, 1, H)

    row3 = pl.BlockSpec((G, 1, H), lambda b: (b, 0, 0))
    full = lambda shape: pl.BlockSpec(shape, lambda b: tuple(0 for _ in shape))

    choice3, nm3, value3, newp, hid3 = pl.pallas_call(
        _main_body,
        grid=(B // G,),
        in_specs=[
            pl.BlockSpec((G, H, H), lambda b: (b, 0, 0)),
            row3, row3, row3,
            full((H, H)),
            full((H, H)),
            full((1, H)),
            full((2, H)),
            full((1, 2)),
            full((1, H)),
            full((1, 1)),
            full((1, H)),
            full((1, 1)),
        ],
        out_specs=[
            pl.BlockSpec((G, 1, 1), lambda b: (b, 0, 0)),
            pl.BlockSpec((G, 1, 1), lambda b: (b, 0, 0)),
            pl.BlockSpec((G, 1, 1), lambda b: (b, 0, 0)),
            pl.BlockSpec((G, H, H), lambda b: (b, 0, 0)),
            row3,
        ],
        out_shape=[
            jax.ShapeDtypeStruct((B, 1, 1), f32),
            jax.ShapeDtypeStruct((B, 1, 1), f32),
            jax.ShapeDtypeStruct((B, 1, 1), f32),
            jax.ShapeDtypeStruct((B, H, H), f32),
            jax.ShapeDtypeStruct((B, 1, H), f32),
        ],
        compiler_params=pltpu.CompilerParams(
            dimension_semantics=("parallel",),
            vmem_limit_bytes=100 * 1024 * 1024),
    )(plastic_weights, emb3, cur3, rew3, alpha, W3, b3.reshape(1, H),
      Wn, bn.reshape(1, 2), Wc, bc.reshape(1, 1), Wv, bv.reshape(1, 1))

    return (choice3.reshape(B, 1), nm3, value3.reshape(B, 1),
            newp, hid3.reshape(B, H))


# speedup vs baseline: 1.0649x; 1.0335x over previous
import jax
import jax.numpy as jnp
from jax.experimental import pallas as pl
from jax.experimental.pallas import tpu as pltpu

_B, _IN, _H = 128, 512, 1024
_G = 2

_NT = (((1,), (1,)), ((), ()))
_OUTER = (((0,), (0,)), ((), ()))


def _main_body(plastic_ref, items_ref, reward_ref, W1_ref, b1_ref, W2_ref,
               b2_ref, Wr_ref, br_ref, alpha_ref, W3_ref, b3_ref, Wn_ref,
               bn_ref, Wc_ref, bc_ref, Wv_ref, bv_ref,
               choice_ref, nm_ref, value_ref, newp_ref, hid_ref,
               emb_s, cur_s, rew_s):
    G, H = _G, _H
    b = pl.program_id(0)

    @pl.when(b == 0)
    def _prologue():
        emb_all = jnp.tanh(
            jax.lax.dot_general(items_ref[...], W1_ref[...], _NT,
                                preferred_element_type=jnp.float32)
            + b1_ref[...])
        emb_s[...] = emb_all.reshape(_B // G, G, H)
        cur_all = jax.lax.dot_general(
            emb_all, W2_ref[...], _NT,
            preferred_element_type=jnp.float32) + b2_ref[...]
        cur_s[...] = cur_all.reshape(_B // G, G, H)
        rew_all = reward_ref[...] * Wr_ref[...] + br_ref[...]
        rew_s[...] = rew_all.reshape(_B // G, G, H)

    alpha = alpha_ref[...]
    emb = emb_s[b]
    pres = []
    for g in range(G):
        M = alpha * plastic_ref[g]
        contrib = jax.lax.dot_general(emb[g:g + 1], M, _NT,
                                      preferred_element_type=jnp.float32)
        pres.append(contrib)
    pre = jnp.concatenate(pres, axis=0) + cur_s[b]
    hidden = jnp.tanh(pre)
    hid_ref[...] = hidden.reshape(G, 1, H)
    t = hidden + rew_s[b]
    h3 = jnp.tanh(
        jax.lax.dot_general(t, W3_ref[...], _NT,
                            preferred_element_type=jnp.float32) + b3_ref[...])
    nmo = jnp.tanh(
        jax.lax.dot_general(h3, Wn_ref[...], _NT,
                            preferred_element_type=jnp.float32) + bn_ref[...])
    nm = nmo[:, 0:1] - nmo[:, 1:2]
    nm_ref[...] = nm.reshape(G, 1, 1)
    choice_ref[...] = jax.nn.sigmoid(
        jnp.sum(hidden * Wc_ref[...], axis=1, keepdims=True)
        + bc_ref[...]).reshape(G, 1, 1)
    value_ref[...] = (jnp.sum(hidden * Wv_ref[...], axis=1, keepdims=True)
                      + bv_ref[...]).reshape(G, 1, 1)
    nm10 = nm * 10.0
    for g in range(G):
        outer = jax.lax.dot_general(pre[g:g + 1], emb[g:g + 1], _OUTER,
                                    preferred_element_type=jnp.float32)
        newp_ref[g] = jnp.clip(
            plastic_ref[g] + nm10[g:g + 1] * jnp.tanh(outer), -50.0, 50.0)


def kernel(items, plastic_weights, reward, W1, b1, W2, b2, W3, b3,
           Wc, bc, Wr, br, Wn, bn, alpha, Wv, bv):
    f32 = jnp.float32
    B, IN, H, G = _B, _IN, _H, _G

    full = lambda shape: pl.BlockSpec(shape, lambda b: tuple(0 for _ in shape))

    choice3, nm3, value3, newp, hid3 = pl.pallas_call(
        _main_body,
        grid=(B // G,),
        in_specs=[
            pl.BlockSpec((G, H, H), lambda b: (b, 0, 0)),
            full((B, IN)),
            full((B, 1)),
            full((H, IN)),
            full((1, H)),
            full((H, H)),
            full((1, H)),
            full((1, H)),
            full((1, H)),
            full((H, H)),
            full((H, H)),
            full((1, H)),
            full((2, H)),
            full((1, 2)),
            full((1, H)),
            full((1, 1)),
            full((1, H)),
            full((1, 1)),
        ],
        out_specs=[
            pl.BlockSpec((G, 1, 1), lambda b: (b, 0, 0)),
            pl.BlockSpec((G, 1, 1), lambda b: (b, 0, 0)),
            pl.BlockSpec((G, 1, 1), lambda b: (b, 0, 0)),
            pl.BlockSpec((G, H, H), lambda b: (b, 0, 0)),
            pl.BlockSpec((G, 1, H), lambda b: (b, 0, 0)),
        ],
        out_shape=[
            jax.ShapeDtypeStruct((B, 1, 1), f32),
            jax.ShapeDtypeStruct((B, 1, 1), f32),
            jax.ShapeDtypeStruct((B, 1, 1), f32),
            jax.ShapeDtypeStruct((B, H, H), f32),
            jax.ShapeDtypeStruct((B, 1, H), f32),
        ],
        scratch_shapes=[
            pltpu.VMEM((B // G, G, H), f32),
            pltpu.VMEM((B // G, G, H), f32),
            pltpu.VMEM((B // G, G, H), f32),
        ],
        compiler_params=pltpu.CompilerParams(
            dimension_semantics=("arbitrary",),
            vmem_limit_bytes=100 * 1024 * 1024),
    )(plastic_weights, items, reward, W1, b1.reshape(1, H), W2,
      b2.reshape(1, H), Wr.reshape(1, H), br.reshape(1, H), alpha, W3,
      b3.reshape(1, H), Wn, bn.reshape(1, 2), Wc, bc.reshape(1, 1),
      Wv, bv.reshape(1, 1))

    return (choice3.reshape(B, 1), nm3, value3.reshape(B, 1),
            newp, hid3.reshape(B, H))
